# baseline (device time: 28777 ns/iter reference)
import jax
import jax.numpy as jnp
from jax import lax
from jax.experimental import pallas as pl
from jax.experimental.pallas import tpu as pltpu

N_DEV = 4


def kernel(Q, K, V):
    b, q_len, h, d = Q.shape
    kk = K.shape[1]
    hd = h * d
    scale = d ** -0.5
    W = hd + 2 * b

    Kt = K.transpose(0, 2, 3, 1).reshape(b, hd, kk)
    Vt = V.transpose(0, 2, 3, 1).reshape(b, hd, kk)
    Q2 = Q.reshape(b * h, d)

    def body(q_ref, k_ref, v_ref, out_ref, kbuf, vbuf, mine_ref, comm_ref,
             ksems, vsems, send_sems, recv_sems):
        my_pos = lax.axis_index("i")

        kdmas, vdmas = [], []
        for bi in range(b):
            kd = pltpu.make_async_copy(k_ref.at[bi], kbuf.at[bi], ksems.at[bi])
            vd = pltpu.make_async_copy(v_ref.at[bi], vbuf.at[bi], vsems.at[bi])
            kd.start()
            vd.start()
            kdmas.append(kd)
            vdmas.append(vd)

        barrier_sem = pltpu.get_barrier_semaphore()
        for j in range(1, N_DEV):
            pl.semaphore_signal(
                barrier_sem, inc=1,
                device_id=((my_pos + j) % N_DEV,),
                device_id_type=pl.DeviceIdType.MESH,
            )

        e2 = (
            lax.broadcasted_iota(jnp.int32, (hd, h), 0) // d
            == lax.broadcasted_iota(jnp.int32, (hd, h), 1)
        ).astype(jnp.float32)
        qt = jnp.swapaxes(q_ref[...], 0, 1)
        edd = (
            lax.broadcasted_iota(jnp.int32, (hd, d), 0) % d
            == lax.broadcasted_iota(jnp.int32, (hd, d), 1)
        ).astype(jnp.float32)
        t0 = jnp.dot(edd, qt, preferred_element_type=jnp.float32)
        mask_h = (
            lax.broadcasted_iota(jnp.int32, (hd, b * h), 0) // d
            == lax.broadcasted_iota(jnp.int32, (hd, b * h), 1) % h
        )
        qbd = jnp.where(mask_h, t0, 0.0) * scale

        for bi in range(b):
            kdmas[bi].wait()
            vdmas[bi].wait()
            s = lax.dot_general(
                kbuf[bi], qbd[:, bi * h:(bi + 1) * h],
                ((( 0,), (0,)), ((), ())),
                preferred_element_type=jnp.float32,
            )
            m_b = jnp.max(s, axis=0, keepdims=True)
            p = jnp.exp(s - m_b)
            l_b = jnp.sum(p, axis=0, keepdims=True)
            x = jnp.dot(vbuf[bi], p,
                        preferred_element_type=jnp.float32)
            acc_b = jnp.sum(x * e2, axis=1, keepdims=True)
            mine_ref[0:hd, bi:bi + 1] = acc_b
            mine_ref[hd + bi:hd + bi + 1, :] = m_b
            mine_ref[hd + b + bi:hd + b + bi + 1, :] = l_b

        pl.semaphore_wait(barrier_sem, N_DEV - 1)

        sends = []
        for j in range(N_DEV - 1):
            rdma = pltpu.make_async_remote_copy(
                src_ref=mine_ref,
                dst_ref=comm_ref.at[2 - j],
                send_sem=send_sems.at[j],
                recv_sem=recv_sems.at[2 - j],
                device_id=((my_pos + 1 + j) % N_DEV,),
                device_id_type=pl.DeviceIdType.MESH,
            )
            rdma.start()
            sends.append(rdma)

        acc_r = mine_ref[0:hd, :]
        m_r = mine_ref[hd:hd + b, :]
        l_r = mine_ref[hd + b:W, :]

        for slot in range(N_DEV - 1):
            recv = pltpu.make_async_remote_copy(
                src_ref=mine_ref,
                dst_ref=comm_ref.at[slot],
                send_sem=send_sems.at[0],
                recv_sem=recv_sems.at[slot],
                device_id=(my_pos,),
                device_id_type=pl.DeviceIdType.MESH,
            )
            recv.wait_recv()
            acc_in = comm_ref[slot, 0:hd, :]
            m_in = comm_ref[slot, hd:hd + b, :]
            l_in = comm_ref[slot, hd + b:W, :]

            m_new = jnp.maximum(m_r, m_in)
            w_r = jnp.exp(m_r - m_new)
            w_in = jnp.exp(m_in - m_new)
            l_r = w_r * l_r + w_in * l_in
            acc_r = (jnp.dot(e2, jnp.swapaxes(w_r, 0, 1),
                             preferred_element_type=jnp.float32) * acc_r
                     + jnp.dot(e2, jnp.swapaxes(w_in, 0, 1),
                               preferred_element_type=jnp.float32) * acc_in)
            m_r = m_new

        for rdma in sends:
            rdma.wait_send()

        out = acc_r / jnp.dot(e2, jnp.swapaxes(l_r, 0, 1),
                              preferred_element_type=jnp.float32)
        out_ref[...] = jnp.swapaxes(out, 0, 1)

    out2 = pl.pallas_call(
        body,
        out_shape=jax.ShapeDtypeStruct((b, hd), jnp.float32),
        in_specs=[
            pl.BlockSpec(memory_space=pltpu.VMEM),
            pl.BlockSpec(memory_space=pl.ANY),
            pl.BlockSpec(memory_space=pl.ANY),
        ],
        out_specs=pl.BlockSpec(memory_space=pltpu.VMEM),
        scratch_shapes=[
            pltpu.VMEM((b, hd, kk), jnp.float32),
            pltpu.VMEM((b, hd, kk), jnp.float32),
            pltpu.VMEM((W, b), jnp.float32),
            pltpu.VMEM((N_DEV - 1, W, b), jnp.float32),
            pltpu.SemaphoreType.DMA((b,)),
            pltpu.SemaphoreType.DMA((b,)),
            pltpu.SemaphoreType.DMA((N_DEV - 1,)),
            pltpu.SemaphoreType.DMA((N_DEV - 1,)),
        ],
        compiler_params=pltpu.CompilerParams(collective_id=0),
    )(Q2, Kt, Vt)
    return out2.reshape(b, q_len, h, d)


# device time: 23151 ns/iter; 1.2430x vs baseline; 1.2430x over previous
import jax
import jax.numpy as jnp
from jax import lax
from jax.experimental import pallas as pl
from jax.experimental.pallas import tpu as pltpu

N_DEV = 4


def kernel(Q, K, V):
    b, q_len, h, d = Q.shape
    kk = K.shape[1]
    hd = h * d
    scale = d ** -0.5
    W = hd + 2 * b

    Kt = K.transpose(0, 2, 3, 1).reshape(b, hd, kk)
    Vt = V.transpose(0, 2, 3, 1).reshape(b, hd, kk)
    Q2 = Q.reshape(b * h, d)

    def body(q_ref, k_ref, v_ref, out_ref, mine_ref, comm_ref,
             send_sems, recv_sems):
        my_pos = lax.axis_index("i")

        barrier_sem = pltpu.get_barrier_semaphore()
        for j in range(1, N_DEV):
            pl.semaphore_signal(
                barrier_sem, inc=1,
                device_id=((my_pos + j) % N_DEV,),
                device_id_type=pl.DeviceIdType.MESH,
            )

        e2 = (
            lax.broadcasted_iota(jnp.int32, (hd, h), 0) // d
            == lax.broadcasted_iota(jnp.int32, (hd, h), 1)
        ).astype(jnp.float32)
        qt = jnp.swapaxes(q_ref[...], 0, 1)
        edd = (
            lax.broadcasted_iota(jnp.int32, (hd, d), 0) % d
            == lax.broadcasted_iota(jnp.int32, (hd, d), 1)
        ).astype(jnp.float32)
        t0 = jnp.dot(edd, qt, preferred_element_type=jnp.float32)
        mask_h = (
            lax.broadcasted_iota(jnp.int32, (hd, b * h), 0) // d
            == lax.broadcasted_iota(jnp.int32, (hd, b * h), 1) % h
        )
        qbd = jnp.where(mask_h, t0, 0.0) * scale

        for bi in range(b):
            s = lax.dot_general(
                k_ref[bi], qbd[:, bi * h:(bi + 1) * h],
                (((0,), (0,)), ((), ())),
                preferred_element_type=jnp.float32,
            )
            m_b = jnp.max(s, axis=0, keepdims=True)
            p = jnp.exp(s - m_b)
            l_b = jnp.sum(p, axis=0, keepdims=True)
            x = jnp.dot(v_ref[bi], p,
                        preferred_element_type=jnp.float32)
            acc_b = jnp.sum(x * e2, axis=1, keepdims=True)
            mine_ref[0:hd, bi:bi + 1] = acc_b
            mine_ref[hd + bi:hd + bi + 1, :] = m_b
            mine_ref[hd + b + bi:hd + b + bi + 1, :] = l_b

        pl.semaphore_wait(barrier_sem, N_DEV - 1)

        sends = []
        for j in range(N_DEV - 1):
            rdma = pltpu.make_async_remote_copy(
                src_ref=mine_ref,
                dst_ref=comm_ref.at[2 - j],
                send_sem=send_sems.at[j],
                recv_sem=recv_sems.at[2 - j],
                device_id=((my_pos + 1 + j) % N_DEV,),
                device_id_type=pl.DeviceIdType.MESH,
            )
            rdma.start()
            sends.append(rdma)

        acc_r = mine_ref[0:hd, :]
        m_r = mine_ref[hd:hd + b, :]
        l_r = mine_ref[hd + b:W, :]

        for slot in range(N_DEV - 1):
            recv = pltpu.make_async_remote_copy(
                src_ref=mine_ref,
                dst_ref=comm_ref.at[slot],
                send_sem=send_sems.at[0],
                recv_sem=recv_sems.at[slot],
                device_id=(my_pos,),
                device_id_type=pl.DeviceIdType.MESH,
            )
            recv.wait_recv()
            acc_in = comm_ref[slot, 0:hd, :]
            m_in = comm_ref[slot, hd:hd + b, :]
            l_in = comm_ref[slot, hd + b:W, :]

            m_new = jnp.maximum(m_r, m_in)
            w_r = jnp.exp(m_r - m_new)
            w_in = jnp.exp(m_in - m_new)
            l_r = w_r * l_r + w_in * l_in
            acc_r = (jnp.dot(e2, jnp.swapaxes(w_r, 0, 1),
                             preferred_element_type=jnp.float32) * acc_r
                     + jnp.dot(e2, jnp.swapaxes(w_in, 0, 1),
                               preferred_element_type=jnp.float32) * acc_in)
            m_r = m_new

        for rdma in sends:
            rdma.wait_send()

        out = acc_r / jnp.dot(e2, jnp.swapaxes(l_r, 0, 1),
                              preferred_element_type=jnp.float32)
        out_ref[...] = jnp.swapaxes(out, 0, 1)

    out2 = pl.pallas_call(
        body,
        out_shape=jax.ShapeDtypeStruct((b, hd), jnp.float32),
        in_specs=[
            pl.BlockSpec(memory_space=pltpu.VMEM),
            pl.BlockSpec(memory_space=pltpu.VMEM),
            pl.BlockSpec(memory_space=pltpu.VMEM),
        ],
        out_specs=pl.BlockSpec(memory_space=pltpu.VMEM),
        scratch_shapes=[
            pltpu.VMEM((W, b), jnp.float32),
            pltpu.VMEM((N_DEV - 1, W, b), jnp.float32),
            pltpu.SemaphoreType.DMA((N_DEV - 1,)),
            pltpu.SemaphoreType.DMA((N_DEV - 1,)),
        ],
        compiler_params=pltpu.CompilerParams(collective_id=0),
    )(Q2, Kt, Vt)
    return out2.reshape(b, q_len, h, d)


# device time: 21919 ns/iter; 1.3129x vs baseline; 1.0562x over previous
import jax
import jax.numpy as jnp
from jax import lax
from jax.experimental import pallas as pl
from jax.experimental.pallas import tpu as pltpu

N_DEV = 4


def kernel(Q, K, V):
    b, q_len, h, d = Q.shape
    kk = K.shape[1]
    hd = h * d
    scale = d ** -0.5
    W = hd + 2 * b

    Kt = K.transpose(0, 2, 3, 1).reshape(b, hd, kk)
    Vt = V.transpose(0, 2, 3, 1).reshape(b, hd, kk)
    Q2 = Q.reshape(b * h, d)

    def body(q_ref, k_ref, v_ref, out_ref, mine_ref, comm_ref,
             send_sems, recv_sems):
        my_pos = lax.axis_index("i")

        barrier_sem = pltpu.get_barrier_semaphore()
        for j in range(1, N_DEV):
            pl.semaphore_signal(
                barrier_sem, inc=1,
                device_id=((my_pos + j) % N_DEV,),
                device_id_type=pl.DeviceIdType.MESH,
            )

        e2 = (
            lax.broadcasted_iota(jnp.int32, (hd, h), 0) // d
            == lax.broadcasted_iota(jnp.int32, (hd, h), 1)
        ).astype(jnp.float32)
        t3 = jnp.dot(
            q_ref[...],
            (lax.broadcasted_iota(jnp.int32, (d, hd), 0)
             == lax.broadcasted_iota(jnp.int32, (d, hd), 1) % d
             ).astype(jnp.float32),
            preferred_element_type=jnp.float32,
        )
        mask_ht = (
            lax.broadcasted_iota(jnp.int32, (b * h, hd), 0) % h
            == lax.broadcasted_iota(jnp.int32, (b * h, hd), 1) // d
        )
        qbt = jnp.where(mask_ht, t3, 0.0) * scale

        for bi in range(b):
            st = jnp.dot(qbt[bi * h:(bi + 1) * h, :], k_ref[bi],
                         preferred_element_type=jnp.float32)
            m_b = jnp.max(st, axis=1, keepdims=True)
            p = jnp.exp(st - m_b)
            l_b = jnp.sum(p, axis=1, keepdims=True)
            x = lax.dot_general(
                v_ref[bi], p,
                (((1,), (1,)), ((), ())),
                preferred_element_type=jnp.float32,
            )
            acc_b = jnp.sum(x * e2, axis=1, keepdims=True)
            mine_ref[0:hd, bi:bi + 1] = acc_b
            mine_ref[hd:hd + h, bi:bi + 1] = m_b
            mine_ref[hd + h:W, bi:bi + 1] = l_b

        pl.semaphore_wait(barrier_sem, N_DEV - 1)

        sends = []
        for j in range(N_DEV - 1):
            rdma = pltpu.make_async_remote_copy(
                src_ref=mine_ref,
                dst_ref=comm_ref.at[2 - j],
                send_sem=send_sems.at[j],
                recv_sem=recv_sems.at[2 - j],
                device_id=((my_pos + 1 + j) % N_DEV,),
                device_id_type=pl.DeviceIdType.MESH,
            )
            rdma.start()
            sends.append(rdma)

        acc_r = mine_ref[0:hd, :]
        m_r = mine_ref[hd:hd + h, :]
        l_r = mine_ref[hd + h:W, :]

        for slot in range(N_DEV - 1):
            recv = pltpu.make_async_remote_copy(
                src_ref=mine_ref,
                dst_ref=comm_ref.at[slot],
                send_sem=send_sems.at[0],
                recv_sem=recv_sems.at[slot],
                device_id=(my_pos,),
                device_id_type=pl.DeviceIdType.MESH,
            )
            recv.wait_recv()
            acc_in = comm_ref[slot, 0:hd, :]
            m_in = comm_ref[slot, hd:hd + h, :]
            l_in = comm_ref[slot, hd + h:W, :]

            m_new = jnp.maximum(m_r, m_in)
            w_r = jnp.exp(m_r - m_new)
            w_in = jnp.exp(m_in - m_new)
            l_r = w_r * l_r + w_in * l_in
            acc_r = (jnp.dot(e2, w_r, preferred_element_type=jnp.float32)
                     * acc_r
                     + jnp.dot(e2, w_in, preferred_element_type=jnp.float32)
                     * acc_in)
            m_r = m_new

        for rdma in sends:
            rdma.wait_send()

        out = acc_r / jnp.dot(e2, l_r, preferred_element_type=jnp.float32)
        out_ref[...] = jnp.swapaxes(out, 0, 1)

    out2 = pl.pallas_call(
        body,
        out_shape=jax.ShapeDtypeStruct((b, hd), jnp.float32),
        in_specs=[
            pl.BlockSpec(memory_space=pltpu.VMEM),
            pl.BlockSpec(memory_space=pltpu.VMEM),
            pl.BlockSpec(memory_space=pltpu.VMEM),
        ],
        out_specs=pl.BlockSpec(memory_space=pltpu.VMEM),
        scratch_shapes=[
            pltpu.VMEM((W, b), jnp.float32),
            pltpu.VMEM((N_DEV - 1, W, b), jnp.float32),
            pltpu.SemaphoreType.DMA((N_DEV - 1,)),
            pltpu.SemaphoreType.DMA((N_DEV - 1,)),
        ],
        compiler_params=pltpu.CompilerParams(collective_id=0),
    )(Q2, Kt, Vt)
    return out2.reshape(b, q_len, h, d)


# device time: 20852 ns/iter; 1.3801x vs baseline; 1.0512x over previous
import jax
import jax.numpy as jnp
from jax import lax
from jax.experimental import pallas as pl
from jax.experimental.pallas import tpu as pltpu

N_DEV = 4


def kernel(Q, K, V):
    b, q_len, h, d = Q.shape
    kk = K.shape[1]
    hd = h * d
    scale = d ** -0.5
    W = hd + 2 * b

    Kt = K.astype(jnp.bfloat16).transpose(0, 2, 3, 1).reshape(b, hd, kk)
    Vt = V.astype(jnp.bfloat16).transpose(0, 2, 3, 1).reshape(b, hd, kk)
    Q2 = Q.reshape(b * h, d)

    def body(q_ref, k_ref, v_ref, out_ref, mine_ref, comm_ref,
             send_sems, recv_sems):
        my_pos = lax.axis_index("i")

        barrier_sem = pltpu.get_barrier_semaphore()
        for j in range(1, N_DEV):
            pl.semaphore_signal(
                barrier_sem, inc=1,
                device_id=((my_pos + j) % N_DEV,),
                device_id_type=pl.DeviceIdType.MESH,
            )

        e2 = (
            lax.broadcasted_iota(jnp.int32, (hd, h), 0) // d
            == lax.broadcasted_iota(jnp.int32, (hd, h), 1)
        ).astype(jnp.float32)
        t3 = jnp.dot(
            q_ref[...],
            (lax.broadcasted_iota(jnp.int32, (d, hd), 0)
             == lax.broadcasted_iota(jnp.int32, (d, hd), 1) % d
             ).astype(jnp.float32),
            preferred_element_type=jnp.float32,
        )
        mask_ht = (
            lax.broadcasted_iota(jnp.int32, (b * h, hd), 0) % h
            == lax.broadcasted_iota(jnp.int32, (b * h, hd), 1) // d
        )
        qbt = (jnp.where(mask_ht, t3, 0.0) * scale).astype(jnp.bfloat16)

        for bi in range(b):
            st = jnp.dot(qbt[bi * h:(bi + 1) * h, :], k_ref[bi],
                         preferred_element_type=jnp.float32)
            m_b = jnp.max(st, axis=1, keepdims=True)
            p = jnp.exp(st - m_b)
            l_b = jnp.sum(p, axis=1, keepdims=True)
            x = lax.dot_general(
                v_ref[bi], p.astype(jnp.bfloat16),
                (((1,), (1,)), ((), ())),
                preferred_element_type=jnp.float32,
            )
            acc_b = jnp.sum(x * e2, axis=1, keepdims=True)
            mine_ref[0:hd, bi:bi + 1] = acc_b
            mine_ref[hd:hd + h, bi:bi + 1] = m_b
            mine_ref[hd + h:W, bi:bi + 1] = l_b

        pl.semaphore_wait(barrier_sem, N_DEV - 1)

        sends = []
        for j in range(N_DEV - 1):
            rdma = pltpu.make_async_remote_copy(
                src_ref=mine_ref,
                dst_ref=comm_ref.at[2 - j],
                send_sem=send_sems.at[j],
                recv_sem=recv_sems.at[2 - j],
                device_id=((my_pos + 1 + j) % N_DEV,),
                device_id_type=pl.DeviceIdType.MESH,
            )
            rdma.start()
            sends.append(rdma)

        acc_r = mine_ref[0:hd, :]
        m_r = mine_ref[hd:hd + h, :]
        l_r = mine_ref[hd + h:W, :]

        for slot in range(N_DEV - 1):
            recv = pltpu.make_async_remote_copy(
                src_ref=mine_ref,
                dst_ref=comm_ref.at[slot],
                send_sem=send_sems.at[0],
                recv_sem=recv_sems.at[slot],
                device_id=(my_pos,),
                device_id_type=pl.DeviceIdType.MESH,
            )
            recv.wait_recv()
            acc_in = comm_ref[slot, 0:hd, :]
            m_in = comm_ref[slot, hd:hd + h, :]
            l_in = comm_ref[slot, hd + h:W, :]

            m_new = jnp.maximum(m_r, m_in)
            w_r = jnp.exp(m_r - m_new)
            w_in = jnp.exp(m_in - m_new)
            l_r = w_r * l_r + w_in * l_in
            acc_r = (jnp.dot(e2, w_r, preferred_element_type=jnp.float32)
                     * acc_r
                     + jnp.dot(e2, w_in, preferred_element_type=jnp.float32)
                     * acc_in)
            m_r = m_new

        for rdma in sends:
            rdma.wait_send()

        out = acc_r / jnp.dot(e2, l_r, preferred_element_type=jnp.float32)
        out_ref[...] = jnp.swapaxes(out, 0, 1)

    out2 = pl.pallas_call(
        body,
        out_shape=jax.ShapeDtypeStruct((b, hd), jnp.float32),
        in_specs=[
            pl.BlockSpec(memory_space=pltpu.VMEM),
            pl.BlockSpec(memory_space=pltpu.VMEM),
            pl.BlockSpec(memory_space=pltpu.VMEM),
        ],
        out_specs=pl.BlockSpec(memory_space=pltpu.VMEM),
        scratch_shapes=[
            pltpu.VMEM((W, b), jnp.float32),
            pltpu.VMEM((N_DEV - 1, W, b), jnp.float32),
            pltpu.SemaphoreType.DMA((N_DEV - 1,)),
            pltpu.SemaphoreType.DMA((N_DEV - 1,)),
        ],
        compiler_params=pltpu.CompilerParams(collective_id=0),
    )(Q2, Kt, Vt)
    return out2.reshape(b, q_len, h, d)


# device time: 15445 ns/iter; 1.8632x vs baseline; 1.3501x over previous
import jax
import jax.numpy as jnp
from jax import lax
from jax.experimental import pallas as pl
from jax.experimental.pallas import tpu as pltpu

N_DEV = 4


def kernel(Q, K, V):
    b, q_len, h, d = Q.shape
    kk = K.shape[1]
    hd = h * d
    scale = d ** -0.5
    W = hd + 2 * b

    Kt = K.astype(jnp.bfloat16).transpose(0, 2, 3, 1).reshape(b, hd, kk)
    Vt = V.astype(jnp.bfloat16).transpose(0, 2, 3, 1).reshape(b, hd, kk)
    Q2 = Q.reshape(b * h, d)

    def body(q_ref, k_ref, v_ref, out_ref, mine_ref, comm_ref,
             send_sems, recv_sems):
        my_pos = lax.axis_index("i")

        barrier_sem = pltpu.get_barrier_semaphore()
        for j in range(1, N_DEV):
            pl.semaphore_signal(
                barrier_sem, inc=1,
                device_id=((my_pos + j) % N_DEV,),
                device_id_type=pl.DeviceIdType.MESH,
            )

        e2t = (
            lax.broadcasted_iota(jnp.int32, (h, hd), 0)
            == lax.broadcasted_iota(jnp.int32, (h, hd), 1) // d
        ).astype(jnp.float32)
        t3 = jnp.dot(
            q_ref[...],
            (lax.broadcasted_iota(jnp.int32, (d, hd), 0)
             == lax.broadcasted_iota(jnp.int32, (d, hd), 1) % d
             ).astype(jnp.float32),
            preferred_element_type=jnp.float32,
        )
        mask_ht = (
            lax.broadcasted_iota(jnp.int32, (b * h, hd), 0) % h
            == lax.broadcasted_iota(jnp.int32, (b * h, hd), 1) // d
        )
        qbt = (jnp.where(mask_ht, t3, 0.0) * scale).astype(jnp.bfloat16)

        bh2 = b // 2

        def compute_batch(bi):
            st = jnp.dot(qbt[bi * h:(bi + 1) * h, :], k_ref[bi],
                         preferred_element_type=jnp.float32)
            m_b = jnp.max(st, axis=1, keepdims=True)
            p = jnp.exp(st - m_b)
            l_b = jnp.sum(p, axis=1, keepdims=True)
            x = lax.dot_general(
                p.astype(jnp.bfloat16), v_ref[bi],
                (((1,), (1,)), ((), ())),
                preferred_element_type=jnp.float32,
            )
            acc_b = jnp.sum(x * e2t, axis=0, keepdims=True)
            half, r = bi // bh2, bi % bh2
            mine_ref[half, r:r + 1, 0:hd] = acc_b
            mine_ref[half, r:r + 1, hd:hd + h] = jnp.swapaxes(m_b, 0, 1)
            mine_ref[half, r:r + 1, hd + h:W] = jnp.swapaxes(l_b, 0, 1)

        def send_half(half):
            out = []
            for j in range(N_DEV - 1):
                rdma = pltpu.make_async_remote_copy(
                    src_ref=mine_ref.at[half],
                    dst_ref=comm_ref.at[2 - j, half],
                    send_sem=send_sems.at[half * 3 + j],
                    recv_sem=recv_sems.at[half * 3 + (2 - j)],
                    device_id=((my_pos + 1 + j) % N_DEV,),
                    device_id_type=pl.DeviceIdType.MESH,
                )
                rdma.start()
                out.append(rdma)
            return out

        for bi in range(bh2):
            compute_batch(bi)
        pl.semaphore_wait(barrier_sem, N_DEV - 1)
        sends = send_half(0)
        for bi in range(bh2, b):
            compute_batch(bi)
        sends += send_half(1)

        for half in range(2):
            acc_r = mine_ref[half, :, 0:hd]
            m_r = mine_ref[half, :, hd:hd + h]
            l_r = mine_ref[half, :, hd + h:W]
            for slot in range(N_DEV - 1):
                recv = pltpu.make_async_remote_copy(
                    src_ref=mine_ref.at[half],
                    dst_ref=comm_ref.at[slot, half],
                    send_sem=send_sems.at[0],
                    recv_sem=recv_sems.at[half * 3 + slot],
                    device_id=(my_pos,),
                    device_id_type=pl.DeviceIdType.MESH,
                )
                recv.wait_recv()
                acc_in = comm_ref[slot, half, :, 0:hd]
                m_in = comm_ref[slot, half, :, hd:hd + h]
                l_in = comm_ref[slot, half, :, hd + h:W]

                m_new = jnp.maximum(m_r, m_in)
                w_r = jnp.exp(m_r - m_new)
                w_in = jnp.exp(m_in - m_new)
                l_r = w_r * l_r + w_in * l_in
                acc_r = (jnp.dot(w_r, e2t, preferred_element_type=jnp.float32)
                         * acc_r
                         + jnp.dot(w_in, e2t,
                                   preferred_element_type=jnp.float32)
                         * acc_in)
                m_r = m_new
            out_ref[half * bh2:(half + 1) * bh2, :] = (
                acc_r / jnp.dot(l_r, e2t, preferred_element_type=jnp.float32)
            )

        for rdma in sends:
            rdma.wait_send()

    out2 = pl.pallas_call(
        body,
        out_shape=jax.ShapeDtypeStruct((b, hd), jnp.float32),
        in_specs=[
            pl.BlockSpec(memory_space=pltpu.VMEM),
            pl.BlockSpec(memory_space=pltpu.VMEM),
            pl.BlockSpec(memory_space=pltpu.VMEM),
        ],
        out_specs=pl.BlockSpec(memory_space=pltpu.VMEM),
        scratch_shapes=[
            pltpu.VMEM((2, b // 2, W), jnp.float32),
            pltpu.VMEM((N_DEV - 1, 2, b // 2, W), jnp.float32),
            pltpu.SemaphoreType.DMA((2 * (N_DEV - 1),)),
            pltpu.SemaphoreType.DMA((2 * (N_DEV - 1),)),
        ],
        compiler_params=pltpu.CompilerParams(collective_id=0),
    )(Q2, Kt, Vt)
    return out2.reshape(b, q_len, h, d)
